# baseline (speedup 1.0000x reference)
import functools

import jax
import jax.numpy as jnp
from jax import lax
from jax.experimental import pallas as pl
from jax.experimental.pallas import tpu as pltpu
from jax.experimental.pallas import tpu_sc as plsc

N_MODALITIES = 8
D_MODEL = 1024

NC = 2
NS = 16
NW = NC * NS

B = 4 * 8192
B_PER_W = B // NW
CHUNK = 32
N_CHUNKS = B_PER_W // CHUNK
NBUF = 3


def _sc_embedding_gather(ids_flat, emb):
    mesh = plsc.VectorSubcoreMesh(
        core_axis_name="c", subcore_axis_name="s", num_cores=NC, num_subcores=NS
    )

    @functools.partial(
        pl.kernel,
        out_type=jax.ShapeDtypeStruct((B, D_MODEL), jnp.float32),
        mesh=mesh,
        scratch_types=[
            pltpu.VMEM((B_PER_W,), jnp.int32),
            [pltpu.VMEM((CHUNK, D_MODEL), jnp.float32) for _ in range(NBUF)],
            [pltpu.SemaphoreType.DMA for _ in range(NBUF)],
            [pltpu.SemaphoreType.DMA for _ in range(NBUF)],
        ],
    )
    def body(idx_hbm, emb_hbm, out_hbm, idx_v, bufs, gsems, ssems):
        cid = lax.axis_index("c")
        sid = lax.axis_index("s")
        wid = sid * NC + cid
        base = wid * B_PER_W

        pltpu.sync_copy(idx_hbm.at[pl.ds(base, B_PER_W)], idx_v)

        def start_gather(g):
            s = g % NBUF
            pltpu.async_copy(
                emb_hbm.at[idx_v.at[pl.ds(g * CHUNK, CHUNK)]], bufs[s], gsems[s]
            )

        def wait_gather(g):
            s = g % NBUF
            pltpu.make_async_copy(
                emb_hbm.at[idx_v.at[pl.ds(g * CHUNK, CHUNK)]], bufs[s], gsems[s]
            ).wait()

        def start_scatter(g):
            s = g % NBUF
            pltpu.async_copy(
                bufs[s], out_hbm.at[pl.ds(base + g * CHUNK, CHUNK)], ssems[s]
            )

        def wait_scatter(g):
            s = g % NBUF
            pltpu.make_async_copy(
                bufs[s], out_hbm.at[pl.ds(base + g * CHUNK, CHUNK)], ssems[s]
            ).wait()

        for g in range(NBUF):
            start_gather(g)
        for g in range(N_CHUNKS):
            wait_gather(g)
            start_scatter(g)
            if g + NBUF < N_CHUNKS:
                wait_scatter(g)
                start_gather(g + NBUF)
        for g in range(N_CHUNKS - NBUF, N_CHUNKS):
            wait_scatter(g)

    return body(ids_flat, emb)


def kernel(modality_ids, emb):
    ids_flat = modality_ids.reshape(-1).astype(jnp.int32)
    out = _sc_embedding_gather(ids_flat, emb)
    return out.reshape(modality_ids.shape + (emb.shape[1],))


# speedup vs baseline: 5.4790x; 5.4790x over previous
import functools

import jax
import jax.numpy as jnp
from jax import lax
from jax.experimental import pallas as pl
from jax.experimental.pallas import tpu as pltpu
from jax.experimental.pallas import tpu_sc as plsc

N_MODALITIES = 8
D_MODEL = 1024

NC = 2
NS = 16
NW = NC * NS

B = 4 * 8192
B_PER_W = B // NW
K = 16
N_BATCH = B_PER_W // K


def _sc_embedding_gather(ids_flat, emb):
    mesh = plsc.VectorSubcoreMesh(
        core_axis_name="c", subcore_axis_name="s", num_cores=NC, num_subcores=NS
    )

    @functools.partial(
        pl.kernel,
        out_type=jax.ShapeDtypeStruct((B, D_MODEL), jnp.float32),
        mesh=mesh,
        scratch_types=[
            pltpu.VMEM((N_MODALITIES, D_MODEL), jnp.float32),
            pltpu.VMEM((B_PER_W,), jnp.int32),
            pltpu.SemaphoreType.DMA,
        ],
    )
    def body(idx_hbm, emb_hbm, out_hbm, tab_v, idx_v, sem):
        cid = lax.axis_index("c")
        sid = lax.axis_index("s")
        wid = sid * NC + cid
        base = wid * B_PER_W

        pltpu.sync_copy(emb_hbm, tab_v)
        pltpu.sync_copy(idx_hbm.at[pl.ds(base, B_PER_W)], idx_v)

        def fire(batch):
            v = idx_v[pl.ds(batch * K, K)]
            for j in range(K):
                rid = v[j]
                pltpu.async_copy(tab_v.at[rid], out_hbm.at[base + batch * K + j], sem)

        def drain():
            for j in range(K):
                pltpu.make_async_copy(tab_v.at[0], out_hbm.at[base], sem).wait()

        fire(0)

        def step(g, _):
            @pl.when(g + 1 < N_BATCH)
            def _():
                fire(g + 1)

            drain()
            return 0

        lax.fori_loop(0, N_BATCH, step, 0)

    return body(ids_flat, emb)


def kernel(modality_ids, emb):
    ids_flat = modality_ids.reshape(-1).astype(jnp.int32)
    out = _sc_embedding_gather(ids_flat, emb)
    return out.reshape(modality_ids.shape + (emb.shape[1],))
